# baseline (speedup 1.0000x reference)
import functools

import jax
import jax.numpy as jnp
from jax import lax
from jax.experimental import pallas as pl
from jax.experimental.pallas import tpu as pltpu
from jax.experimental.pallas import tpu_sc as plsc

N_NODES = 10000
N_EDGES = 320000
D = 128
DW = D // 2
DE = 16
LANES = 16

NC = 2
NS = 16
NW = NC * NS
EPW = N_EDGES // NW
CHUNK = 200
NCHUNK = EPW // CHUNK

_SC_MESH = plsc.VectorSubcoreMesh(
    core_axis_name="c", subcore_axis_name="s", num_cores=NC, num_subcores=NS)


def _pack_bf16_halves(y):
    yb = y.astype(jnp.bfloat16)
    lo = lax.bitcast_convert_type(yb[:, :D // 2], jnp.uint16).astype(jnp.uint32)
    hi = lax.bitcast_convert_type(yb[:, D // 2:], jnp.uint16).astype(jnp.uint32)
    return lax.bitcast_convert_type(lo | (hi << 16), jnp.int32)


def _tables_body(x_ref, ws_ref, wr_ref, xs_ref, xr_ref):
    xs_ref[...] = _pack_bf16_halves(
        jnp.dot(x_ref[...], ws_ref[...], preferred_element_type=jnp.float32))
    xr_ref[...] = _pack_bf16_halves(
        jnp.dot(x_ref[...], wr_ref[...], preferred_element_type=jnp.float32))


def _combine_body(g_ref, ea_ref, we_ref, b_ref, o_ref):
    g = g_ref[...]
    lo = lax.bitcast_convert_type(lax.shift_left(g, 16), jnp.float32)
    hi = lax.bitcast_convert_type(g & jnp.int32(-65536), jnp.float32)
    o_ref[...] = (jnp.concatenate([lo, hi], axis=1)
                  + jnp.dot(ea_ref[...], we_ref[...],
                            preferred_element_type=jnp.float32)
                  + b_ref[...])


def _gather_sum_body(xs_hbm, xr_hbm, sidx_hbm, ridx_hbm, out_hbm,
                     sidx_v, ridx_v, a0, a1, b0, b1, o0, o1,
                     sa0, sa1, sb0, sb1, so0, so1):
    wid = lax.axis_index("s") * NC + lax.axis_index("c")
    base = wid * EPW
    pltpu.sync_copy(sidx_hbm.at[pl.ds(base, EPW)], sidx_v)
    pltpu.sync_copy(ridx_hbm.at[pl.ds(base, EPW)], ridx_v)

    a = (a0, a1)
    b = (b0, b1)
    o = (o0, o1)
    sa = (sa0, sa1)
    sb = (sb0, sb1)
    so = (so0, so1)

    def gather(c, k):
        off = c * CHUNK
        pltpu.async_copy(xs_hbm.at[sidx_v.at[pl.ds(off, CHUNK)]], a[k], sa[k])
        pltpu.async_copy(xr_hbm.at[ridx_v.at[pl.ds(off, CHUNK)]], b[k], sb[k])

    def wait_gather(k):
        pltpu.make_async_copy(xs_hbm.at[pl.ds(0, CHUNK)], a[k], sa[k]).wait()
        pltpu.make_async_copy(xr_hbm.at[pl.ds(0, CHUNK)], b[k], sb[k]).wait()

    def wait_write(k):
        pltpu.make_async_copy(o[k], out_hbm.at[pl.ds(0, CHUNK)], so[k]).wait()

    gather(0, 0)
    gather(1, 1)

    def pair_body(p, carry):
        for k in range(2):
            c = p * 2 + k
            wait_gather(k)

            @pl.when(c >= 2)
            def _():
                wait_write(k)

            def row_body(i, carry2):
                hi_mask = jnp.int32(-65536)
                rnd = jnp.int32(0x8000)
                for j in range(DW // LANES):
                    sl = pl.ds(j * LANES, LANES)
                    av = a[k][i, sl]
                    bv = b[k][i, sl]
                    lo = (plsc.bitcast(lax.shift_left(av, 16), jnp.float32)
                          + plsc.bitcast(lax.shift_left(bv, 16), jnp.float32))
                    hi = (plsc.bitcast(av & hi_mask, jnp.float32)
                          + plsc.bitcast(bv & hi_mask, jnp.float32))
                    lo_b = lax.shift_right_logical(
                        plsc.bitcast(lo, jnp.int32) + rnd, 16)
                    hi_b = (plsc.bitcast(hi, jnp.int32) + rnd) & hi_mask
                    o[k][i, sl] = lo_b | hi_b
                return carry2

            lax.fori_loop(0, CHUNK, row_body, 0, unroll=2)

            pltpu.async_copy(o[k], out_hbm.at[pl.ds(base + c * CHUNK, CHUNK)],
                             so[k])

            @pl.when(c + 2 < NCHUNK)
            def _():
                gather(c + 2, k)

        return carry

    lax.fori_loop(0, NCHUNK // 2, pair_body, 0)
    wait_write(0)
    wait_write(1)


_gather_sum = pl.kernel(
    _gather_sum_body,
    out_type=jax.ShapeDtypeStruct((N_EDGES, DW), jnp.int32),
    mesh=_SC_MESH,
    compiler_params=pltpu.CompilerParams(use_tc_tiling_on_sc=False,
                                         needs_layout_passes=False),
    scratch_types=[
        pltpu.VMEM((EPW,), jnp.int32),
        pltpu.VMEM((EPW,), jnp.int32),
        pltpu.VMEM((CHUNK, DW), jnp.int32),
        pltpu.VMEM((CHUNK, DW), jnp.int32),
        pltpu.VMEM((CHUNK, DW), jnp.int32),
        pltpu.VMEM((CHUNK, DW), jnp.int32),
        pltpu.VMEM((CHUNK, DW), jnp.int32),
        pltpu.VMEM((CHUNK, DW), jnp.int32),
        pltpu.SemaphoreType.DMA,
        pltpu.SemaphoreType.DMA,
        pltpu.SemaphoreType.DMA,
        pltpu.SemaphoreType.DMA,
        pltpu.SemaphoreType.DMA,
        pltpu.SemaphoreType.DMA,
    ],
)

_EB = 3200


def kernel(x, edge_index, edge_attr, W, b):
    s_idx = edge_index[0].astype(jnp.int32)
    r_idx = edge_index[1].astype(jnp.int32)
    ws = W[:D]
    wr = W[D:2 * D]
    we = W[2 * D:]
    b2 = b.reshape(1, D)

    xs, xr = pl.pallas_call(
        _tables_body,
        out_shape=[jax.ShapeDtypeStruct((N_NODES, DW), jnp.int32)] * 2,
    )(x, ws, wr)

    g = _gather_sum(xs, xr, s_idx, r_idx)

    out = pl.pallas_call(
        _combine_body,
        grid=(N_EDGES // _EB,),
        in_specs=[
            pl.BlockSpec((_EB, DW), lambda i: (i, 0)),
            pl.BlockSpec((_EB, DE), lambda i: (i, 0)),
            pl.BlockSpec((DE, D), lambda i: (0, 0)),
            pl.BlockSpec((1, D), lambda i: (0, 0)),
        ],
        out_specs=pl.BlockSpec((_EB, D), lambda i: (i, 0)),
        out_shape=jax.ShapeDtypeStruct((N_EDGES, D), jnp.float32),
    )(g, edge_attr, we, b2)

    return (x, edge_index, out)
